# baseline (speedup 1.0000x reference)
import functools

import jax
import jax.numpy as jnp
from jax import lax
from jax.experimental import pallas as pl
from jax.experimental.pallas import tpu as pltpu
from jax.experimental.pallas import tpu_sc as plsc

_B, _N, _C = 8, 8192, 64
_S, _K = 1024, 32
_R2 = 0.2 * 0.2
_COUT = 128


_NCH = _N // 128


def _fps_body(xyz_ref, cx_ref, cy_ref, cz_ref, dists_ref):
    iota_l = lax.broadcasted_iota(jnp.int32, (_B, 128), 1)
    big = jnp.full((_B, 128), 1e10, jnp.float32)
    for j in range(_NCH):
        dists_ref[:, j * 128:(j + 1) * 128] = big

    def blk_body(blk, carry):

        def it_body(i2, carry2):
            cx, cy, cz, a128x, a128y, a128z = carry2
            eql = iota_l == i2
            a128x = jnp.where(eql, cx, a128x)
            a128y = jnp.where(eql, cy, a128y)
            a128z = jnp.where(eql, cz, a128z)
            rm = [jnp.full((_B, 128), -1.0, jnp.float32)] * 4
            ridx = [jnp.zeros((_B, 128), jnp.int32)] * 4
            zc = jnp.zeros((_B, 128), jnp.float32)
            rx = [zc] * 4
            ry = [zc] * 4
            rz = [zc] * 4
            for j in range(_NCH):
                sl = slice(j * 128, (j + 1) * 128)
                k = j % 4
                xj = xyz_ref[0, :, sl]
                yj = xyz_ref[1, :, sl]
                zj = xyz_ref[2, :, sl]
                dx = xj - cx
                dy = yj - cy
                dz = zj - cz
                d = dx * dx + dy * dy + dz * dz
                dj = jnp.minimum(dists_ref[:, sl], d)
                dists_ref[:, sl] = dj
                upd = dj > rm[k]
                ridx[k] = jnp.where(upd, iota_l + j * 128, ridx[k])
                rx[k] = jnp.where(upd, xj, rx[k])
                ry[k] = jnp.where(upd, yj, ry[k])
                rz[k] = jnp.where(upd, zj, rz[k])
                rm[k] = jnp.maximum(rm[k], dj)
            finv = jnp.maximum(
                jnp.maximum(rm[0], rm[1]), jnp.maximum(rm[2], rm[3])
            )
            fin = jnp.max(finv, axis=1, keepdims=True)
            cands = [
                jnp.where(rm[k] == fin, ridx[k], _N) for k in range(4)
            ]
            cand = jnp.minimum(
                jnp.minimum(cands[0], cands[1]), jnp.minimum(cands[2], cands[3])
            )
            far = jnp.min(cand, axis=1, keepdims=True)
            sx = zc
            sy = zc
            sz = zc
            for k in range(4):
                eqw = cands[k] == far
                sx = sx + jnp.where(eqw, rx[k], 0.0)
                sy = sy + jnp.where(eqw, ry[k], 0.0)
                sz = sz + jnp.where(eqw, rz[k], 0.0)
            cx = jnp.sum(sx, axis=1, keepdims=True)
            cy = jnp.sum(sy, axis=1, keepdims=True)
            cz = jnp.sum(sz, axis=1, keepdims=True)
            return cx, cy, cz, a128x, a128y, a128z

        cx, cy, cz, a128x, a128y, a128z = lax.fori_loop(0, 128, it_body, carry)
        base = pl.multiple_of(blk * 128, 128)
        cx_ref[:, pl.ds(base, 128)] = a128x
        cy_ref[:, pl.ds(base, 128)] = a128y
        cz_ref[:, pl.ds(base, 128)] = a128z
        return cx, cy, cz, a128x, a128y, a128z

    z128 = jnp.zeros((_B, 128), jnp.float32)
    lax.fori_loop(
        0,
        _S // 128,
        blk_body,
        (
            xyz_ref[0, :, 0:1],
            xyz_ref[1, :, 0:1],
            xyz_ref[2, :, 0:1],
            z128,
            z128,
            z128,
        ),
    )


def _fps(xyz_t):
    out = jax.ShapeDtypeStruct((_B, _S), jnp.float32)
    return pl.pallas_call(
        _fps_body,
        out_shape=[out, out, out],
        scratch_shapes=[pltpu.VMEM((_B, _N), jnp.float32)],
    )(xyz_t)


_NW = 32
_CPW = _B * _S // _NW
_QPB = _NW // _B
_NCHUNK = _N // 16


_UNROLL = 4
_NGRP = _N // (16 * _UNROLL)


def _bq_body(
    x_hbm, y_hbm, z_hbm, cx_hbm, cy_hbm, cz_hbm, feats_hbm,
    gx_hbm, gf_hbm,
    xv, yv, zv, cxv, cyv, czv, idxbuf,
    idxf0, idxf1, rows0, rows1, gxbuf0, gxbuf1,
    gsem0, gsem1, gfsem0, gfsem1, gxsem0, gxsem1,
):
    wid = lax.axis_index("s") * 2 + lax.axis_index("c")
    b = wid // _QPB
    q = wid % _QPB
    pltpu.sync_copy(x_hbm.at[pl.ds(b * _N, _N)], xv)
    pltpu.sync_copy(y_hbm.at[pl.ds(b * _N, _N)], yv)
    pltpu.sync_copy(z_hbm.at[pl.ds(b * _N, _N)], zv)
    pltpu.sync_copy(cx_hbm.at[pl.ds(b * _S, _S)], cxv)
    pltpu.sync_copy(cy_hbm.at[pl.ds(b * _S, _S)], cyv)
    pltpu.sync_copy(cz_hbm.at[pl.ds(b * _S, _S)], czv)

    zf = jnp.zeros((16,), jnp.float32)
    for j in range(_K * 8 // 16):
        gxbuf0[j * 16:(j + 1) * 16] = zf
        gxbuf1[j * 16:(j + 1) * 16] = zf
    lanes = lax.iota(jnp.int32, 16)
    zeros_i = jnp.zeros((16,), jnp.int32)
    r2 = jnp.float32(_R2)

    sets = (
        (idxf0, rows0, gxbuf0, gsem0, gfsem0, gxsem0),
        (idxf1, rows1, gxbuf1, gsem1, gfsem1, gxsem1),
    )

    def gc_of(g):
        return b * _S + g * _QPB + q

    def finish(g_prev, st):
        idxf, rows, gxbuf, gsem, gfsem, gxsem = st
        pltpu.make_async_copy(feats_hbm.at[idxf], rows, gsem).wait()
        gc = gc_of(g_prev)
        pltpu.async_copy(rows, gf_hbm.at[pl.ds(gc * _K, _K)], gfsem)
        pltpu.async_copy(gxbuf, gx_hbm.at[pl.ds(gc * _K * 8, _K * 8)], gxsem)

    def phase(g, cur, prv):
        idxf, rows, gxbuf, gsem, gfsem, gxsem = cur
        sidx = g * _QPB + q
        gi = jnp.full((16,), sidx, jnp.int32)
        ccx = plsc.load_gather(cxv, [gi])
        ccy = plsc.load_gather(cyv, [gi])
        ccz = plsc.load_gather(czv, [gi])
        idxbuf[0:16] = zeros_i

        def cond(st):
            cnt, j = st
            return (cnt < _K) & (j < _NGRP)

        def scan_body(st):
            cnt, j = st
            off = j * (16 * _UNROLL)
            ms = []
            for u in range(_UNROLL):
                o2 = off + u * 16
                px = xv[pl.ds(o2, 16)]
                py = yv[pl.ds(o2, 16)]
                pz = zv[pl.ds(o2, 16)]
                dx = px - ccx
                dy = py - ccy
                dz = pz - ccz
                d2 = dx * dx + dy * dy + dz * dz
                ms.append((o2, d2 <= r2))
            c0 = cnt
            for o2, m in ms:
                plsc.store_compressed(idxbuf.at[pl.ds(c0, 16)], o2 + lanes, mask=m)
                c0 = c0 + jnp.sum(jnp.where(m, jnp.int32(1), jnp.int32(0)))
            return c0, j + jnp.int32(1)

        cnt, _ = lax.while_loop(cond, scan_body, (jnp.int32(0), jnp.int32(0)))

        @pl.when(g >= 2)
        def _():
            gc2 = gc_of(g - 2)
            pltpu.make_async_copy(
                rows, gf_hbm.at[pl.ds(gc2 * _K, _K)], gfsem
            ).wait()
            pltpu.make_async_copy(
                gxbuf, gx_hbm.at[pl.ds(gc2 * _K * 8, _K * 8)], gxsem
            ).wait()

        first = plsc.load_gather(idxbuf, [zeros_i])
        cntv = jnp.full((16,), cnt, jnp.int32)
        sels = []
        for s in range(_K // 16):
            pos = lanes + s * 16
            cur_idx = idxbuf[s * 16:(s + 1) * 16]
            sel = jnp.where(pos < cntv, cur_idx, first)
            sels.append((pos, sel))
            idxf[s * 16:(s + 1) * 16] = sel + b * _N
        pltpu.async_copy(feats_hbm.at[idxf], rows, gsem)

        for pos, sel in sels:
            relx = plsc.load_gather(xv, [sel]) - ccx
            rely = plsc.load_gather(yv, [sel]) - ccy
            relz = plsc.load_gather(zv, [sel]) - ccz
            rowbase = pos * 8
            plsc.store_scatter(gxbuf, [rowbase], relx)
            plsc.store_scatter(gxbuf, [rowbase + 1], rely)
            plsc.store_scatter(gxbuf, [rowbase + 2], relz)

        @pl.when(g >= 1)
        def _():
            finish(g - 1, prv)

    def outer_body(o, carry):
        phase(o * 2, sets[0], sets[1])
        phase(o * 2 + 1, sets[1], sets[0])
        return carry

    lax.fori_loop(0, _CPW // 2, outer_body, jnp.int32(0))

    last = _CPW - 1
    finish(last, sets[1])
    for g_done, st in ((last - 1, sets[0]), (last, sets[1])):
        idxf, rows, gxbuf, gsem, gfsem, gxsem = st
        gc = gc_of(g_done)
        pltpu.make_async_copy(rows, gf_hbm.at[pl.ds(gc * _K, _K)], gfsem).wait()
        pltpu.make_async_copy(
            gxbuf, gx_hbm.at[pl.ds(gc * _K * 8, _K * 8)], gxsem
        ).wait()


def _ball_group(x, y, z, cx, cy, cz, feats_t):
    mesh = plsc.VectorSubcoreMesh(core_axis_name="c", subcore_axis_name="s")
    kfn = functools.partial(
        pl.kernel,
        mesh=mesh,
        out_type=[
            jax.ShapeDtypeStruct((_B * _S * _K * 8,), jnp.float32),
            jax.ShapeDtypeStruct((_B * _S * _K, _C), jnp.float32),
        ],
        scratch_types=[
            pltpu.VMEM((_N,), jnp.float32),
            pltpu.VMEM((_N,), jnp.float32),
            pltpu.VMEM((_N,), jnp.float32),
            pltpu.VMEM((_S,), jnp.float32),
            pltpu.VMEM((_S,), jnp.float32),
            pltpu.VMEM((_S,), jnp.float32),
            pltpu.VMEM((128,), jnp.int32),
            pltpu.VMEM((_K,), jnp.int32),
            pltpu.VMEM((_K,), jnp.int32),
            pltpu.VMEM((_K, _C), jnp.float32),
            pltpu.VMEM((_K, _C), jnp.float32),
            pltpu.VMEM((_K * 8,), jnp.float32),
            pltpu.VMEM((_K * 8,), jnp.float32),
            pltpu.SemaphoreType.DMA,
            pltpu.SemaphoreType.DMA,
            pltpu.SemaphoreType.DMA,
            pltpu.SemaphoreType.DMA,
            pltpu.SemaphoreType.DMA,
            pltpu.SemaphoreType.DMA,
        ],
        compiler_params=pltpu.CompilerParams(
            needs_layout_passes=False, use_tc_tiling_on_sc=False
        ),
    )(_bq_body)
    return kfn(x, y, z, cx, cy, cz, feats_t)


_TS = 128


def _mlp_body(gx_ref, gf_ref, w0x_ref, w0f_ref, b0_ref, w1_ref, b1_ref, out_ref):
    h = jnp.dot(gx_ref[...], w0x_ref[...], preferred_element_type=jnp.float32)
    h = h + jnp.dot(gf_ref[...], w0f_ref[...], preferred_element_type=jnp.float32)
    h = jnp.maximum(h + b0_ref[...], 0.0)
    h2 = jnp.dot(h, w1_ref[...], preferred_element_type=jnp.float32)
    h2 = jnp.maximum(h2 + b1_ref[...], 0.0)
    p = jnp.max(h2.reshape(_TS, _K, _COUT), axis=1)
    out_ref[0] = p.T


def _mlp(gx, gf, w0x, w0f, b0, w1, b1):
    nsi = _S // _TS
    grid = (_B, nsi)
    return pl.pallas_call(
        _mlp_body,
        grid=grid,
        in_specs=[
            pl.BlockSpec((_TS * _K, 8), lambda bi, si: (bi * nsi + si, 0)),
            pl.BlockSpec((_TS * _K, _C), lambda bi, si: (bi * nsi + si, 0)),
            pl.BlockSpec((8, _C), lambda bi, si: (0, 0)),
            pl.BlockSpec((_C, _C), lambda bi, si: (0, 0)),
            pl.BlockSpec((1, _C), lambda bi, si: (0, 0)),
            pl.BlockSpec((_C, _COUT), lambda bi, si: (0, 0)),
            pl.BlockSpec((1, _COUT), lambda bi, si: (0, 0)),
        ],
        out_specs=pl.BlockSpec((1, _COUT, _TS), lambda bi, si: (bi, 0, si)),
        out_shape=jax.ShapeDtypeStruct((_B, _COUT, _S), jnp.float32),
    )(gx, gf, w0x, w0f, b0, w1, b1)


def kernel(xyz, features, W0, b0, W1, b1):
    xyz_t = jnp.transpose(xyz, (2, 0, 1))
    cx, cy, cz = _fps(xyz_t)
    new_xyz = jnp.stack([cx, cy, cz], axis=-1)

    feats_t = jnp.transpose(features, (0, 2, 1)).reshape(_B * _N, _C)
    x = xyz_t[0].reshape(_B * _N)
    y = xyz_t[1].reshape(_B * _N)
    z = xyz_t[2].reshape(_B * _N)
    gx, gf = _ball_group(
        x, y, z, cx.reshape(-1), cy.reshape(-1), cz.reshape(-1), feats_t
    )
    gx = gx.reshape(_B * _S * _K, 8)

    w0x = jnp.zeros((8, _C), jnp.float32).at[:3, :].set(W0[:, :3].T)
    w0f = W0[:, 3:].T
    w1 = W1.T
    new_features = _mlp(
        gx, gf, w0x, w0f, b0.reshape(1, _C), w1, b1.reshape(1, _COUT)
    )
    return new_xyz, new_features


# speedup vs baseline: 1.0521x; 1.0521x over previous
import functools

import jax
import jax.numpy as jnp
from jax import lax
from jax.experimental import pallas as pl
from jax.experimental.pallas import tpu as pltpu
from jax.experimental.pallas import tpu_sc as plsc

_B, _N, _C = 8, 8192, 64
_S, _K = 1024, 32
_R2 = 0.2 * 0.2
_COUT = 128


_NCH = _N // 128


def _fps_body(xyz_ref, cx_ref, cy_ref, cz_ref, dists_ref):
    iota_l = lax.broadcasted_iota(jnp.int32, (_B, 128), 1)
    big = jnp.full((_B, 128), 1e10, jnp.float32)
    for j in range(_NCH):
        dists_ref[:, j * 128:(j + 1) * 128] = big

    def blk_body(blk, carry):

        def it_body(i2, carry2):
            far, a128x, a128y, a128z = carry2
            zc = jnp.zeros((_B, 128), jnp.float32)
            ax = [zc] * 4
            ay = [zc] * 4
            az = [zc] * 4
            for j in range(_NCH):
                sl = slice(j * 128, (j + 1) * 128)
                k = j % 4
                eq = iota_l == (far - j * 128)
                ax[k] = ax[k] + jnp.where(eq, xyz_ref[0, :, sl], 0.0)
                ay[k] = ay[k] + jnp.where(eq, xyz_ref[1, :, sl], 0.0)
                az[k] = az[k] + jnp.where(eq, xyz_ref[2, :, sl], 0.0)
            cx = jnp.sum((ax[0] + ax[1]) + (ax[2] + ax[3]), axis=1, keepdims=True)
            cy = jnp.sum((ay[0] + ay[1]) + (ay[2] + ay[3]), axis=1, keepdims=True)
            cz = jnp.sum((az[0] + az[1]) + (az[2] + az[3]), axis=1, keepdims=True)
            eql = iota_l == i2
            a128x = jnp.where(eql, cx, a128x)
            a128y = jnp.where(eql, cy, a128y)
            a128z = jnp.where(eql, cz, a128z)
            rm = [jnp.full((_B, 128), -1.0, jnp.float32)] * 4
            ridx = [jnp.zeros((_B, 128), jnp.int32)] * 4
            for j in range(_NCH):
                sl = slice(j * 128, (j + 1) * 128)
                k = j % 4
                dx = xyz_ref[0, :, sl] - cx
                dy = xyz_ref[1, :, sl] - cy
                dz = xyz_ref[2, :, sl] - cz
                d = (dx * dx + dz * dz) + dy * dy
                dj = jnp.minimum(dists_ref[:, sl], d)
                dists_ref[:, sl] = dj
                upd = dj > rm[k]
                ridx[k] = jnp.where(upd, iota_l + j * 128, ridx[k])
                rm[k] = jnp.maximum(rm[k], dj)
            finv = jnp.maximum(
                jnp.maximum(rm[0], rm[1]), jnp.maximum(rm[2], rm[3])
            )
            fin = jnp.max(finv, axis=1, keepdims=True)
            cands = [
                jnp.where(rm[k] == fin, ridx[k], _N) for k in range(4)
            ]
            cand = jnp.minimum(
                jnp.minimum(cands[0], cands[1]), jnp.minimum(cands[2], cands[3])
            )
            far = jnp.min(cand, axis=1, keepdims=True)
            return far, a128x, a128y, a128z

        far, a128x, a128y, a128z = lax.fori_loop(0, 128, it_body, carry)
        base = pl.multiple_of(blk * 128, 128)
        cx_ref[:, pl.ds(base, 128)] = a128x
        cy_ref[:, pl.ds(base, 128)] = a128y
        cz_ref[:, pl.ds(base, 128)] = a128z
        return far, a128x, a128y, a128z

    z128 = jnp.zeros((_B, 128), jnp.float32)
    lax.fori_loop(
        0,
        _S // 128,
        blk_body,
        (jnp.zeros((_B, 1), jnp.int32), z128, z128, z128),
    )


def _fps(xyz_t):
    out = jax.ShapeDtypeStruct((_B, _S), jnp.float32)
    return pl.pallas_call(
        _fps_body,
        out_shape=[out, out, out],
        scratch_shapes=[pltpu.VMEM((_B, _N), jnp.float32)],
    )(xyz_t)


_NW = 32
_CPW = _B * _S // _NW
_QPB = _NW // _B
_NCHUNK = _N // 16


_UNROLL = 4
_NGRP = _N // (16 * _UNROLL)


def _bq_body(
    x_hbm, y_hbm, z_hbm, cx_hbm, cy_hbm, cz_hbm, feats_hbm,
    gx_hbm, gf_hbm,
    xv, yv, zv, cxv, cyv, czv, idxbuf,
    idxf0, idxf1, rows0, rows1, gxbuf0, gxbuf1,
    gsem0, gsem1, gfsem0, gfsem1, gxsem0, gxsem1,
):
    wid = lax.axis_index("s") * 2 + lax.axis_index("c")
    b = wid // _QPB
    q = wid % _QPB
    pltpu.sync_copy(x_hbm.at[pl.ds(b * _N, _N)], xv)
    pltpu.sync_copy(y_hbm.at[pl.ds(b * _N, _N)], yv)
    pltpu.sync_copy(z_hbm.at[pl.ds(b * _N, _N)], zv)
    pltpu.sync_copy(cx_hbm.at[pl.ds(b * _S, _S)], cxv)
    pltpu.sync_copy(cy_hbm.at[pl.ds(b * _S, _S)], cyv)
    pltpu.sync_copy(cz_hbm.at[pl.ds(b * _S, _S)], czv)

    zf = jnp.zeros((16,), jnp.float32)
    for j in range(_K * 8 // 16):
        gxbuf0[j * 16:(j + 1) * 16] = zf
        gxbuf1[j * 16:(j + 1) * 16] = zf
    lanes = lax.iota(jnp.int32, 16)
    zeros_i = jnp.zeros((16,), jnp.int32)
    r2 = jnp.float32(_R2)

    sets = (
        (idxf0, rows0, gxbuf0, gsem0, gfsem0, gxsem0),
        (idxf1, rows1, gxbuf1, gsem1, gfsem1, gxsem1),
    )

    def gc_of(g):
        return b * _S + g * _QPB + q

    def finish(g_prev, st):
        idxf, rows, gxbuf, gsem, gfsem, gxsem = st
        pltpu.make_async_copy(feats_hbm.at[idxf], rows, gsem).wait()
        gc = gc_of(g_prev)
        pltpu.async_copy(rows, gf_hbm.at[pl.ds(gc * _K, _K)], gfsem)
        pltpu.async_copy(gxbuf, gx_hbm.at[pl.ds(gc * _K * 8, _K * 8)], gxsem)

    def phase(g, cur, prv):
        idxf, rows, gxbuf, gsem, gfsem, gxsem = cur
        sidx = g * _QPB + q
        gi = jnp.full((16,), sidx, jnp.int32)
        ccx = plsc.load_gather(cxv, [gi])
        ccy = plsc.load_gather(cyv, [gi])
        ccz = plsc.load_gather(czv, [gi])
        idxbuf[0:16] = zeros_i

        def cond(st):
            cnt, j = st
            return (cnt < _K) & (j < _NGRP)

        def scan_body(st):
            cnt, j = st
            off = j * (16 * _UNROLL)
            ms = []
            for u in range(_UNROLL):
                o2 = off + u * 16
                px = xv[pl.ds(o2, 16)]
                py = yv[pl.ds(o2, 16)]
                pz = zv[pl.ds(o2, 16)]
                dx = px - ccx
                dy = py - ccy
                dz = pz - ccz
                d2 = dx * dx + dy * dy + dz * dz
                ms.append((o2, d2 <= r2))
            c0 = cnt
            for o2, m in ms:
                plsc.store_compressed(idxbuf.at[pl.ds(c0, 16)], o2 + lanes, mask=m)
                c0 = c0 + jnp.sum(jnp.where(m, jnp.int32(1), jnp.int32(0)))
            return c0, j + jnp.int32(1)

        cnt, _ = lax.while_loop(cond, scan_body, (jnp.int32(0), jnp.int32(0)))

        @pl.when(g >= 2)
        def _():
            gc2 = gc_of(g - 2)
            pltpu.make_async_copy(
                rows, gf_hbm.at[pl.ds(gc2 * _K, _K)], gfsem
            ).wait()
            pltpu.make_async_copy(
                gxbuf, gx_hbm.at[pl.ds(gc2 * _K * 8, _K * 8)], gxsem
            ).wait()

        first = plsc.load_gather(idxbuf, [zeros_i])
        cntv = jnp.full((16,), cnt, jnp.int32)
        sels = []
        for s in range(_K // 16):
            pos = lanes + s * 16
            cur_idx = idxbuf[s * 16:(s + 1) * 16]
            sel = jnp.where(pos < cntv, cur_idx, first)
            sels.append((pos, sel))
            idxf[s * 16:(s + 1) * 16] = sel + b * _N
        pltpu.async_copy(feats_hbm.at[idxf], rows, gsem)

        for pos, sel in sels:
            relx = plsc.load_gather(xv, [sel]) - ccx
            rely = plsc.load_gather(yv, [sel]) - ccy
            relz = plsc.load_gather(zv, [sel]) - ccz
            rowbase = pos * 8
            plsc.store_scatter(gxbuf, [rowbase], relx)
            plsc.store_scatter(gxbuf, [rowbase + 1], rely)
            plsc.store_scatter(gxbuf, [rowbase + 2], relz)

        @pl.when(g >= 1)
        def _():
            finish(g - 1, prv)

    def outer_body(o, carry):
        phase(o * 2, sets[0], sets[1])
        phase(o * 2 + 1, sets[1], sets[0])
        return carry

    lax.fori_loop(0, _CPW // 2, outer_body, jnp.int32(0))

    last = _CPW - 1
    finish(last, sets[1])
    for g_done, st in ((last - 1, sets[0]), (last, sets[1])):
        idxf, rows, gxbuf, gsem, gfsem, gxsem = st
        gc = gc_of(g_done)
        pltpu.make_async_copy(rows, gf_hbm.at[pl.ds(gc * _K, _K)], gfsem).wait()
        pltpu.make_async_copy(
            gxbuf, gx_hbm.at[pl.ds(gc * _K * 8, _K * 8)], gxsem
        ).wait()


def _ball_group(x, y, z, cx, cy, cz, feats_t):
    mesh = plsc.VectorSubcoreMesh(core_axis_name="c", subcore_axis_name="s")
    kfn = functools.partial(
        pl.kernel,
        mesh=mesh,
        out_type=[
            jax.ShapeDtypeStruct((_B * _S * _K * 8,), jnp.float32),
            jax.ShapeDtypeStruct((_B * _S * _K, _C), jnp.float32),
        ],
        scratch_types=[
            pltpu.VMEM((_N,), jnp.float32),
            pltpu.VMEM((_N,), jnp.float32),
            pltpu.VMEM((_N,), jnp.float32),
            pltpu.VMEM((_S,), jnp.float32),
            pltpu.VMEM((_S,), jnp.float32),
            pltpu.VMEM((_S,), jnp.float32),
            pltpu.VMEM((128,), jnp.int32),
            pltpu.VMEM((_K,), jnp.int32),
            pltpu.VMEM((_K,), jnp.int32),
            pltpu.VMEM((_K, _C), jnp.float32),
            pltpu.VMEM((_K, _C), jnp.float32),
            pltpu.VMEM((_K * 8,), jnp.float32),
            pltpu.VMEM((_K * 8,), jnp.float32),
            pltpu.SemaphoreType.DMA,
            pltpu.SemaphoreType.DMA,
            pltpu.SemaphoreType.DMA,
            pltpu.SemaphoreType.DMA,
            pltpu.SemaphoreType.DMA,
            pltpu.SemaphoreType.DMA,
        ],
        compiler_params=pltpu.CompilerParams(
            needs_layout_passes=False, use_tc_tiling_on_sc=False
        ),
    )(_bq_body)
    return kfn(x, y, z, cx, cy, cz, feats_t)


_TS = 128


def _mlp_body(gx_ref, gf_ref, w0x_ref, w0f_ref, b0_ref, w1_ref, b1_ref, out_ref):
    h = jnp.dot(gx_ref[...], w0x_ref[...], preferred_element_type=jnp.float32)
    h = h + jnp.dot(gf_ref[...], w0f_ref[...], preferred_element_type=jnp.float32)
    h = jnp.maximum(h + b0_ref[...], 0.0)
    h2 = jnp.dot(h, w1_ref[...], preferred_element_type=jnp.float32)
    h2 = jnp.maximum(h2 + b1_ref[...], 0.0)
    p = jnp.max(h2.reshape(_TS, _K, _COUT), axis=1)
    out_ref[0] = p.T


def _mlp(gx, gf, w0x, w0f, b0, w1, b1):
    nsi = _S // _TS
    grid = (_B, nsi)
    return pl.pallas_call(
        _mlp_body,
        grid=grid,
        in_specs=[
            pl.BlockSpec((_TS * _K, 8), lambda bi, si: (bi * nsi + si, 0)),
            pl.BlockSpec((_TS * _K, _C), lambda bi, si: (bi * nsi + si, 0)),
            pl.BlockSpec((8, _C), lambda bi, si: (0, 0)),
            pl.BlockSpec((_C, _C), lambda bi, si: (0, 0)),
            pl.BlockSpec((1, _C), lambda bi, si: (0, 0)),
            pl.BlockSpec((_C, _COUT), lambda bi, si: (0, 0)),
            pl.BlockSpec((1, _COUT), lambda bi, si: (0, 0)),
        ],
        out_specs=pl.BlockSpec((1, _COUT, _TS), lambda bi, si: (bi, 0, si)),
        out_shape=jax.ShapeDtypeStruct((_B, _COUT, _S), jnp.float32),
    )(gx, gf, w0x, w0f, b0, w1, b1)


def kernel(xyz, features, W0, b0, W1, b1):
    xyz_t = jnp.transpose(xyz, (2, 0, 1))
    cx, cy, cz = _fps(xyz_t)
    new_xyz = jnp.stack([cx, cy, cz], axis=-1)

    feats_t = jnp.transpose(features, (0, 2, 1)).reshape(_B * _N, _C)
    x = xyz_t[0].reshape(_B * _N)
    y = xyz_t[1].reshape(_B * _N)
    z = xyz_t[2].reshape(_B * _N)
    gx, gf = _ball_group(
        x, y, z, cx.reshape(-1), cy.reshape(-1), cz.reshape(-1), feats_t
    )
    gx = gx.reshape(_B * _S * _K, 8)

    w0x = jnp.zeros((8, _C), jnp.float32).at[:3, :].set(W0[:, :3].T)
    w0f = W0[:, 3:].T
    w1 = W1.T
    new_features = _mlp(
        gx, gf, w0x, w0f, b0.reshape(1, _C), w1, b1.reshape(1, _COUT)
    )
    return new_xyz, new_features


# speedup vs baseline: 1.2947x; 1.2306x over previous
import functools

import jax
import jax.numpy as jnp
from jax import lax
from jax.experimental import pallas as pl
from jax.experimental.pallas import tpu as pltpu
from jax.experimental.pallas import tpu_sc as plsc

_B, _N, _C = 8, 8192, 64
_S, _K = 1024, 32
_R2 = 0.2 * 0.2
_COUT = 128


_NCH = _N // 128


def _fps_body(xyz_ref, cx_ref, cy_ref, cz_ref, dists_ref):
    iota_l = lax.broadcasted_iota(jnp.int32, (_B, 128), 1)
    big = jnp.full((_B, 128), 1e10, jnp.float32)
    for j in range(_NCH):
        dists_ref[:, j * 128:(j + 1) * 128] = big

    def blk_body(blk, carry):

        def it_body(i2, carry2):
            far, a128x, a128y, a128z = carry2
            zc = jnp.zeros((_B, 128), jnp.float32)
            ax = [zc] * 4
            ay = [zc] * 4
            az = [zc] * 4
            for j in range(_NCH):
                sl = slice(j * 128, (j + 1) * 128)
                k = j % 4
                eq = iota_l == (far - j * 128)
                ax[k] = ax[k] + jnp.where(eq, xyz_ref[0, :, sl], 0.0)
                ay[k] = ay[k] + jnp.where(eq, xyz_ref[1, :, sl], 0.0)
                az[k] = az[k] + jnp.where(eq, xyz_ref[2, :, sl], 0.0)
            cx = jnp.sum((ax[0] + ax[1]) + (ax[2] + ax[3]), axis=1, keepdims=True)
            cy = jnp.sum((ay[0] + ay[1]) + (ay[2] + ay[3]), axis=1, keepdims=True)
            cz = jnp.sum((az[0] + az[1]) + (az[2] + az[3]), axis=1, keepdims=True)
            eql = iota_l == i2
            a128x = jnp.where(eql, cx, a128x)
            a128y = jnp.where(eql, cy, a128y)
            a128z = jnp.where(eql, cz, a128z)
            rm = [jnp.full((_B, 128), -1.0, jnp.float32)] * 4
            ridx = [jnp.zeros((_B, 128), jnp.int32)] * 4
            for j in range(_NCH):
                sl = slice(j * 128, (j + 1) * 128)
                k = j % 4
                dx = xyz_ref[0, :, sl] - cx
                dy = xyz_ref[1, :, sl] - cy
                dz = xyz_ref[2, :, sl] - cz
                d = (dx * dx + dz * dz) + dy * dy
                dj = jnp.minimum(dists_ref[:, sl], d)
                dists_ref[:, sl] = dj
                upd = dj > rm[k]
                ridx[k] = jnp.where(upd, iota_l + j * 128, ridx[k])
                rm[k] = jnp.maximum(rm[k], dj)
            finv = jnp.maximum(
                jnp.maximum(rm[0], rm[1]), jnp.maximum(rm[2], rm[3])
            )
            fin = jnp.max(finv, axis=1, keepdims=True)
            cands = [
                jnp.where(rm[k] == fin, ridx[k], _N) for k in range(4)
            ]
            cand = jnp.minimum(
                jnp.minimum(cands[0], cands[1]), jnp.minimum(cands[2], cands[3])
            )
            far = jnp.min(cand, axis=1, keepdims=True)
            return far, a128x, a128y, a128z

        far, a128x, a128y, a128z = lax.fori_loop(0, 128, it_body, carry)
        base = pl.multiple_of(blk * 128, 128)
        cx_ref[:, pl.ds(base, 128)] = a128x
        cy_ref[:, pl.ds(base, 128)] = a128y
        cz_ref[:, pl.ds(base, 128)] = a128z
        return far, a128x, a128y, a128z

    z128 = jnp.zeros((_B, 128), jnp.float32)
    lax.fori_loop(
        0,
        _S // 128,
        blk_body,
        (jnp.zeros((_B, 1), jnp.int32), z128, z128, z128),
    )


def _fps(xyz_t):
    out = jax.ShapeDtypeStruct((_B, _S), jnp.float32)
    return pl.pallas_call(
        _fps_body,
        out_shape=[out, out, out],
        scratch_shapes=[pltpu.VMEM((_B, _N), jnp.float32)],
    )(xyz_t)


_NW = 32
_CPW = _B * _S // _NW
_QPB = _NW // _B
_NCHUNK = _N // 16


_UNROLL = 4
_NGRP = _N // (16 * _UNROLL)


def _bq_body(
    x_hbm, y_hbm, z_hbm, cx_hbm, cy_hbm, cz_hbm, feats_hbm,
    gf_hbm,
    xv, yv, zv, cxv, cyv, czv, idxbuf,
    idxf0, idxf1, rows0, rows1,
    gsem0, gsem1, gfsem0, gfsem1,
):
    wid = lax.axis_index("s") * 2 + lax.axis_index("c")
    b = wid // _QPB
    q = wid % _QPB
    pltpu.sync_copy(x_hbm.at[pl.ds(b * _N, _N)], xv)
    pltpu.sync_copy(y_hbm.at[pl.ds(b * _N, _N)], yv)
    pltpu.sync_copy(z_hbm.at[pl.ds(b * _N, _N)], zv)
    pltpu.sync_copy(cx_hbm.at[pl.ds(b * _S, _S)], cxv)
    pltpu.sync_copy(cy_hbm.at[pl.ds(b * _S, _S)], cyv)
    pltpu.sync_copy(cz_hbm.at[pl.ds(b * _S, _S)], czv)

    lanes = lax.iota(jnp.int32, 16)
    zeros_i = jnp.zeros((16,), jnp.int32)
    r2 = jnp.float32(_R2)

    sets = (
        (idxf0, rows0, gsem0, gfsem0),
        (idxf1, rows1, gsem1, gfsem1),
    )

    def gc_of(g):
        return b * _S + g * _QPB + q

    def finish(g_prev, st):
        idxf, rows, gsem, gfsem = st
        pltpu.make_async_copy(feats_hbm.at[idxf], rows, gsem).wait()
        sidx = g_prev * _QPB + q
        gi = jnp.full((16,), sidx, jnp.int32)
        ccx = plsc.load_gather(cxv, [gi])
        ccy = plsc.load_gather(cyv, [gi])
        ccz = plsc.load_gather(czv, [gi])
        for s in range(_K // 16):
            pos = lanes + s * 16
            sel = idxf[s * 16:(s + 1) * 16] - b * _N
            relx = plsc.load_gather(xv, [sel]) - ccx
            rely = plsc.load_gather(yv, [sel]) - ccy
            relz = plsc.load_gather(zv, [sel]) - ccz
            c0 = jnp.zeros((16,), jnp.int32)
            plsc.store_scatter(rows, [pos, c0], relx)
            plsc.store_scatter(rows, [pos, c0 + 1], rely)
            plsc.store_scatter(rows, [pos, c0 + 2], relz)
        gc = gc_of(g_prev)
        pltpu.async_copy(rows, gf_hbm.at[pl.ds(gc * _K, _K)], gfsem)

    def phase(g, cur, prv):
        idxf, rows, gsem, gfsem = cur
        sidx = g * _QPB + q
        gi = jnp.full((16,), sidx, jnp.int32)
        ccx = plsc.load_gather(cxv, [gi])
        ccy = plsc.load_gather(cyv, [gi])
        ccz = plsc.load_gather(czv, [gi])
        idxbuf[0:16] = zeros_i

        def cond(st):
            cnt, j = st
            return (cnt < _K) & (j < _NGRP)

        def scan_body(st):
            cnt, j = st
            off = j * (16 * _UNROLL)
            ms = []
            for u in range(_UNROLL):
                o2 = off + u * 16
                px = xv[pl.ds(o2, 16)]
                py = yv[pl.ds(o2, 16)]
                pz = zv[pl.ds(o2, 16)]
                dx = px - ccx
                dy = py - ccy
                dz = pz - ccz
                d2 = (dx * dx + dz * dz) + dy * dy
                ms.append((o2, d2 <= r2))
            c0 = cnt
            for o2, m in ms:
                plsc.store_compressed(idxbuf.at[pl.ds(c0, 16)], o2 + lanes, mask=m)
                c0 = c0 + jnp.sum(jnp.where(m, jnp.int32(1), jnp.int32(0)))
            return c0, j + jnp.int32(1)

        cnt, _ = lax.while_loop(cond, scan_body, (jnp.int32(0), jnp.int32(0)))

        @pl.when(g >= 2)
        def _():
            gc2 = gc_of(g - 2)
            pltpu.make_async_copy(
                rows, gf_hbm.at[pl.ds(gc2 * _K, _K)], gfsem
            ).wait()

        first = plsc.load_gather(idxbuf, [zeros_i])
        cntv = jnp.full((16,), cnt, jnp.int32)
        for s in range(_K // 16):
            pos = lanes + s * 16
            cur_idx = idxbuf[s * 16:(s + 1) * 16]
            sel = jnp.where(pos < cntv, cur_idx, first)
            idxf[s * 16:(s + 1) * 16] = sel + b * _N
        pltpu.async_copy(feats_hbm.at[idxf], rows, gsem)

        @pl.when(g >= 1)
        def _():
            finish(g - 1, prv)

    def outer_body(o, carry):
        phase(o * 2, sets[0], sets[1])
        phase(o * 2 + 1, sets[1], sets[0])
        return carry

    lax.fori_loop(0, _CPW // 2, outer_body, jnp.int32(0))

    last = _CPW - 1
    finish(last, sets[1])
    for g_done, st in ((last - 1, sets[0]), (last, sets[1])):
        idxf, rows, gsem, gfsem = st
        gc = gc_of(g_done)
        pltpu.make_async_copy(rows, gf_hbm.at[pl.ds(gc * _K, _K)], gfsem).wait()


def _ball_group(x, y, z, cx, cy, cz, table):
    mesh = plsc.VectorSubcoreMesh(core_axis_name="c", subcore_axis_name="s")
    kfn = functools.partial(
        pl.kernel,
        mesh=mesh,
        out_type=jax.ShapeDtypeStruct((_B * _S * _K, 128), jnp.float32),
        scratch_types=[
            pltpu.VMEM((_N,), jnp.float32),
            pltpu.VMEM((_N,), jnp.float32),
            pltpu.VMEM((_N,), jnp.float32),
            pltpu.VMEM((_S,), jnp.float32),
            pltpu.VMEM((_S,), jnp.float32),
            pltpu.VMEM((_S,), jnp.float32),
            pltpu.VMEM((128,), jnp.int32),
            pltpu.VMEM((_K,), jnp.int32),
            pltpu.VMEM((_K,), jnp.int32),
            pltpu.VMEM((_K, 128), jnp.float32),
            pltpu.VMEM((_K, 128), jnp.float32),
            pltpu.SemaphoreType.DMA,
            pltpu.SemaphoreType.DMA,
            pltpu.SemaphoreType.DMA,
            pltpu.SemaphoreType.DMA,
        ],
        compiler_params=pltpu.CompilerParams(needs_layout_passes=False),
    )(_bq_body)
    return kfn(x, y, z, cx, cy, cz, table)


_TS = 128


def _mlp_body(gf_ref, w0_ref, b0_ref, w1_ref, b1_ref, out_ref):
    h = jnp.dot(gf_ref[...], w0_ref[...], preferred_element_type=jnp.float32)
    h = jnp.maximum(h + b0_ref[...], 0.0)
    h2 = jnp.dot(h, w1_ref[...], preferred_element_type=jnp.float32)
    h2 = jnp.maximum(h2 + b1_ref[...], 0.0)
    p = jnp.max(h2.reshape(_TS, _K, _COUT), axis=1)
    out_ref[0] = p.T


def _mlp(gf, w0, b0, w1, b1):
    nsi = _S // _TS
    grid = (_B, nsi)
    return pl.pallas_call(
        _mlp_body,
        grid=grid,
        in_specs=[
            pl.BlockSpec((_TS * _K, 128), lambda bi, si: (bi * nsi + si, 0)),
            pl.BlockSpec((128, _C), lambda bi, si: (0, 0)),
            pl.BlockSpec((1, _C), lambda bi, si: (0, 0)),
            pl.BlockSpec((_C, _COUT), lambda bi, si: (0, 0)),
            pl.BlockSpec((1, _COUT), lambda bi, si: (0, 0)),
        ],
        out_specs=pl.BlockSpec((1, _COUT, _TS), lambda bi, si: (bi, 0, si)),
        out_shape=jax.ShapeDtypeStruct((_B, _COUT, _S), jnp.float32),
    )(gf, w0, b0, w1, b1)


def kernel(xyz, features, W0, b0, W1, b1):
    xyz_t = jnp.transpose(xyz, (2, 0, 1))
    cx, cy, cz = _fps(xyz_t)
    new_xyz = jnp.stack([cx, cy, cz], axis=-1)

    feats_t = jnp.transpose(features, (0, 2, 1)).reshape(_B * _N, _C)
    table = (
        jnp.zeros((_B * _N, 128), jnp.float32).at[:, 64:].set(feats_t)
    )
    x = xyz_t[0].reshape(_B * _N)
    y = xyz_t[1].reshape(_B * _N)
    z = xyz_t[2].reshape(_B * _N)
    gf = _ball_group(
        x, y, z, cx.reshape(-1), cy.reshape(-1), cz.reshape(-1), table
    )

    w0 = (
        jnp.zeros((128, _C), jnp.float32)
        .at[:3, :].set(W0[:, :3].T)
        .at[64:, :].set(W0[:, 3:].T)
    )
    w1 = W1.T
    new_features = _mlp(gf, w0, b0.reshape(1, _C), w1, b1.reshape(1, _COUT))
    return new_xyz, new_features
